# Initial kernel scaffold; baseline (speedup 1.0000x reference)
import jax
import jax.numpy as jnp
from jax.experimental import pallas as pl


def kernel(in_features, W_v, W_n, gamma, beta, alpha, reduce_index, gather_index):
    raise NotImplementedError("write your pallas kernel here")


# speedup vs baseline: 7.8756x; 7.8756x over previous
import functools

import jax
import jax.numpy as jnp
from jax import lax
from jax.experimental import pallas as pl
from jax.experimental.pallas import tpu as pltpu
from jax.experimental.pallas import tpu_sc as plsc

N_NODES = 10000
C = 128
D = 144
E = 320000
EPS = 1e-5

NC, NS, L = 2, 16, 16
NW = NC * NS
EPT = E // NW
CHUNK = 40
NCHUNK = EPT // CHUNK
N_PAD = 10000
STRIPE = N_PAD // NS


_sc_mesh = plsc.VectorSubcoreMesh(
    core_axis_name="c", subcore_axis_name="s", num_cores=NC, num_subcores=NS
)


def _sc_body(xaug, g_hbm, r_hbm, out, acc, gidx, ridx, rows0, rows1, sem0, sem1):
    c = lax.axis_index("c")
    s = lax.axis_index("s")
    wid = c * NS + s

    zv = jnp.zeros((L,), jnp.float32)

    def zrow(i, carry):
        rows0[i // (D // L), pl.ds((i % (D // L)) * L, L)] = zv
        return carry

    lax.fori_loop(0, CHUNK * (D // L), zrow, 0)
    for k in range(STRIPE // CHUNK):
        pltpu.sync_copy(rows0, acc.at[pl.ds(s * STRIPE + k * CHUNK, CHUNK)])
    rem = STRIPE % CHUNK
    if rem:
        pltpu.sync_copy(
            rows0.at[pl.ds(0, rem)],
            acc.at[pl.ds(s * STRIPE + (STRIPE // CHUNK) * CHUNK, rem)],
        )
    plsc.subcore_barrier()

    pltpu.sync_copy(g_hbm.at[wid], gidx)
    pltpu.sync_copy(r_hbm.at[wid], ridx)

    pltpu.async_copy(xaug.at[gidx.at[0]], rows0, sem0)
    pltpu.async_copy(xaug.at[gidx.at[1]], rows1, sem1)

    def pair(j, carry):
        a = 2 * j
        pltpu.make_async_copy(xaug.at[gidx.at[0]], rows0, sem0).wait()
        pltpu.sync_copy(rows0, acc.at[ridx.at[a]], add=True)

        @pl.when(a + 2 < NCHUNK)
        def _():
            pltpu.async_copy(xaug.at[gidx.at[a + 2]], rows0, sem0)

        pltpu.make_async_copy(xaug.at[gidx.at[1]], rows1, sem1).wait()
        pltpu.sync_copy(rows1, acc.at[ridx.at[a + 1]], add=True)

        @pl.when(a + 3 < NCHUNK)
        def _():
            pltpu.async_copy(xaug.at[gidx.at[a + 3]], rows1, sem1)

        return carry

    lax.fori_loop(0, NCHUNK // 2, pair, 0)

    plsc.subcore_barrier()
    pltpu.sync_copy(
        acc.at[pl.ds(s * STRIPE, STRIPE)],
        out.at[c, pl.ds(s * STRIPE, STRIPE)],
    )


_sc_aggregate = functools.partial(
    pl.kernel,
    out_type=jax.ShapeDtypeStruct((NC, N_PAD, D), jnp.float32),
    mesh=_sc_mesh,
    compiler_params=pltpu.CompilerParams(use_tc_tiling_on_sc=False),
    scratch_types=[
        pltpu.VMEM_SHARED((N_PAD, D), jnp.float32),
        pltpu.VMEM((NCHUNK, CHUNK), jnp.int32),
        pltpu.VMEM((NCHUNK, CHUNK), jnp.int32),
        pltpu.VMEM((CHUNK, D), jnp.float32),
        pltpu.VMEM((CHUNK, D), jnp.float32),
        pltpu.SemaphoreType.DMA,
        pltpu.SemaphoreType.DMA,
    ],
)(_sc_body)


def _tc_body(x_ref, p_ref, wv_ref, wn_ref, g_ref, b_ref, a_ref, o_ref):
    x = x_ref[...]
    sums = p_ref[0, :N_NODES, :C] + p_ref[1, :N_NODES, :C]
    counts = p_ref[0, :N_NODES, C:C + 1] + p_ref[1, :N_NODES, C:C + 1]
    mean = sums * (1.0 / jnp.maximum(counts, 1.0))
    zn = lax.dot_general(wn_ref[...], mean, (((1,), (1,)), ((), ())),
                         preferred_element_type=jnp.float32)
    zv = lax.dot_general(wv_ref[...], x, (((1,), (0,)), ((), ())),
                         preferred_element_type=jnp.float32)
    z = zv + zn
    mu = jnp.mean(z, axis=1, keepdims=True)
    var = jnp.mean(z * z, axis=1, keepdims=True) - mu * mu
    scale = g_ref[...] * lax.rsqrt(var + EPS)
    out = (z - mu) * scale + b_ref[...]
    alpha = a_ref[0, 0]
    o_ref[...] = jnp.where(out >= 0, out, alpha * out)


_tc_finish = pl.pallas_call(
    _tc_body,
    out_shape=jax.ShapeDtypeStruct((C, N_NODES), jnp.float32),
    in_specs=[
        pl.BlockSpec(memory_space=pltpu.VMEM),
        pl.BlockSpec(memory_space=pltpu.VMEM),
        pl.BlockSpec(memory_space=pltpu.VMEM),
        pl.BlockSpec(memory_space=pltpu.VMEM),
        pl.BlockSpec(memory_space=pltpu.VMEM),
        pl.BlockSpec(memory_space=pltpu.VMEM),
        pl.BlockSpec(memory_space=pltpu.SMEM),
    ],
)


def kernel(in_features, W_v, W_n, gamma, beta, alpha, reduce_index, gather_index):
    x = in_features[0]
    xaug = jnp.concatenate(
        [x.T, jnp.ones((N_NODES, 1), jnp.float32),
         jnp.zeros((N_NODES, D - C - 1), jnp.float32)], axis=1)
    g3 = gather_index.reshape(NW, NCHUNK, CHUNK)
    r3 = reduce_index.reshape(NW, NCHUNK, CHUNK)
    partials = _sc_aggregate(xaug, g3, r3)
    out = _tc_finish(
        x, partials, W_v, W_n,
        gamma.reshape(C, 1), beta.reshape(C, 1),
        jnp.reshape(alpha, (1, 1)),
    )
    return out[None]


# speedup vs baseline: 12.9331x; 1.6422x over previous
import functools

import jax
import jax.numpy as jnp
from jax import lax
from jax.experimental import pallas as pl
from jax.experimental.pallas import tpu as pltpu
from jax.experimental.pallas import tpu_sc as plsc

N_NODES = 10000
C = 128
E = 320000
EPS = 1e-5

NC, NS, L = 2, 16, 16
NW = NC * NS
EPT = E // NW
CHUNK = 40
NCHUNK = EPT // CHUNK
NBUF = 3
N_PAD = 10240
STRIPE = N_PAD // NS


_sc_mesh = plsc.VectorSubcoreMesh(
    core_axis_name="c", subcore_axis_name="s", num_cores=NC, num_subcores=NS
)


def _sc_body(xt, g_hbm, r_hbm, out_s, out_c, acc, cnt, gidx, ridx,
             rows0, rows1, rows2, rb0, rb1, rb2,
             gs0, gs1, gs2, ss0, ss1, ss2):
    c = lax.axis_index("c")
    s = lax.axis_index("s")
    wid = c * NS + s
    ROWS = (rows0, rows1, rows2)
    RB = (rb0, rb1, rb2)
    GS = (gs0, gs1, gs2)
    SS = (ss0, ss1, ss2)

    zv = jnp.zeros((L,), jnp.float32)

    def zrow(i, carry):
        rows0[i // (C // L), pl.ds((i % (C // L)) * L, L)] = zv
        return carry

    lax.fori_loop(0, CHUNK * (C // L), zrow, 0)
    for k in range(STRIPE // CHUNK):
        pltpu.sync_copy(rows0, acc.at[pl.ds(s * STRIPE + k * CHUNK, CHUNK)])

    def zcnt(i, carry):
        cnt[pl.ds(i * L, L)] = zv
        return carry

    lax.fori_loop(0, N_PAD // L, zcnt, 0)
    plsc.subcore_barrier()

    pltpu.sync_copy(g_hbm.at[wid], gidx)
    pltpu.sync_copy(r_hbm.at[wid], ridx)

    def g_start(a, b):
        pltpu.async_copy(xt.at[gidx.at[pl.ds(a * CHUNK, CHUNK)]], ROWS[b], GS[b])

    def g_wait(b):
        pltpu.make_async_copy(xt.at[gidx.at[pl.ds(0, CHUNK)]], ROWS[b], GS[b]).wait()

    def s_wait(b):
        pltpu.make_async_copy(ROWS[b], acc.at[RB[b]], SS[b]).wait()

    tail = lax.iota(jnp.int32, L) >= (2 * L - (CHUNK - L))

    def process(a, b):
        base = a * CHUNK
        v0 = ridx[pl.ds(base, L)]
        v1 = ridx[pl.ds(base + L, L)]
        v2 = ridx[pl.ds(base + CHUNK - L, L)]
        RB[b][pl.ds(0, L)] = v0
        RB[b][pl.ds(L, L)] = v1
        RB[b][pl.ds(CHUNK - L, L)] = v2
        r0, l0 = plsc.scan_count(v0)
        plsc.addupdate_scatter(cnt, [v0], r0.astype(jnp.float32), mask=l0)
        r1, l1 = plsc.scan_count(v1)
        plsc.addupdate_scatter(cnt, [v1], r1.astype(jnp.float32), mask=l1)
        r2, l2 = plsc.scan_count(v2, mask=tail)
        plsc.addupdate_scatter(cnt, [v2], r2.astype(jnp.float32), mask=l2 & tail)
        pltpu.async_copy(ROWS[b], acc.at[RB[b]], SS[b], add=True)

    g_start(0, 0)
    g_start(1, 1)

    def blocks(j, carry):
        for b in range(NBUF):
            a = NBUF * j + b

            @pl.when(a < NCHUNK)
            def _():
                g_wait(b)
                process(a, b)
                b2 = (b + 2) % NBUF

                @pl.when(a + 2 < NCHUNK)
                def _():
                    @pl.when(a >= 1)
                    def _():
                        s_wait(b2)

                    g_start(a + 2, b2)

        return carry

    lax.fori_loop(0, (NCHUNK + NBUF - 1) // NBUF, blocks, 0)
    s_wait(0)
    s_wait(1)
    s_wait(2)

    plsc.subcore_barrier()
    pltpu.sync_copy(
        acc.at[pl.ds(s * STRIPE, STRIPE)],
        out_s.at[c, pl.ds(s * STRIPE, STRIPE)],
    )
    pltpu.sync_copy(cnt, out_c.at[c, s])


_sc_aggregate = functools.partial(
    pl.kernel,
    out_type=(
        jax.ShapeDtypeStruct((NC, N_PAD, C), jnp.float32),
        jax.ShapeDtypeStruct((NC, NS, N_PAD), jnp.float32),
    ),
    mesh=_sc_mesh,
    compiler_params=pltpu.CompilerParams(needs_layout_passes=False),
    scratch_types=[
        pltpu.VMEM_SHARED((N_PAD, C), jnp.float32),
        pltpu.VMEM((N_PAD,), jnp.float32),
        pltpu.VMEM((EPT,), jnp.int32),
        pltpu.VMEM((EPT,), jnp.int32),
        pltpu.VMEM((CHUNK, C), jnp.float32),
        pltpu.VMEM((CHUNK, C), jnp.float32),
        pltpu.VMEM((CHUNK, C), jnp.float32),
        pltpu.VMEM((CHUNK,), jnp.int32),
        pltpu.VMEM((CHUNK,), jnp.int32),
        pltpu.VMEM((CHUNK,), jnp.int32),
        pltpu.SemaphoreType.DMA,
        pltpu.SemaphoreType.DMA,
        pltpu.SemaphoreType.DMA,
        pltpu.SemaphoreType.DMA,
        pltpu.SemaphoreType.DMA,
        pltpu.SemaphoreType.DMA,
    ],
)(_sc_body)


def _tc_body(x_ref, p_ref, c_ref, wv_ref, wn_ref, g_ref, b_ref, a_ref, o_ref):
    x = x_ref[...]
    sums = p_ref[0, :N_NODES, :] + p_ref[1, :N_NODES, :]
    counts = jnp.sum(c_ref[...], axis=(0, 1))[:N_NODES]
    inv = 1.0 / jnp.maximum(counts, 1.0)
    zn = lax.dot_general(wn_ref[...], sums, (((1,), (1,)), ((), ())),
                         preferred_element_type=jnp.float32)
    zv = lax.dot_general(wv_ref[...], x, (((1,), (0,)), ((), ())),
                         preferred_element_type=jnp.float32)
    z = zv + zn * inv[None, :]
    mu = jnp.mean(z, axis=1, keepdims=True)
    var = jnp.mean(z * z, axis=1, keepdims=True) - mu * mu
    scale = g_ref[...] * lax.rsqrt(var + EPS)
    out = (z - mu) * scale + b_ref[...]
    alpha = a_ref[0, 0]
    o_ref[...] = jnp.where(out >= 0, out, alpha * out)


_tc_finish = pl.pallas_call(
    _tc_body,
    out_shape=jax.ShapeDtypeStruct((C, N_NODES), jnp.float32),
    in_specs=[
        pl.BlockSpec(memory_space=pltpu.VMEM),
        pl.BlockSpec(memory_space=pltpu.VMEM),
        pl.BlockSpec(memory_space=pltpu.VMEM),
        pl.BlockSpec(memory_space=pltpu.VMEM),
        pl.BlockSpec(memory_space=pltpu.VMEM),
        pl.BlockSpec(memory_space=pltpu.VMEM),
        pl.BlockSpec(memory_space=pltpu.VMEM),
        pl.BlockSpec(memory_space=pltpu.SMEM),
    ],
)


def kernel(in_features, W_v, W_n, gamma, beta, alpha, reduce_index, gather_index):
    x = in_features[0]
    xt = x.T
    g2 = gather_index.reshape(NW, EPT)
    r2 = reduce_index.reshape(NW, EPT)
    psums, pcnts = _sc_aggregate(xt, g2, r2)
    out = _tc_finish(
        x, psums, pcnts, W_v, W_n,
        gamma.reshape(C, 1), beta.reshape(C, 1),
        jnp.reshape(alpha, (1, 1)),
    )
    return out[None]


# speedup vs baseline: 15.2626x; 1.1801x over previous
import functools

import jax
import jax.numpy as jnp
from jax import lax
from jax.experimental import pallas as pl
from jax.experimental.pallas import tpu as pltpu
from jax.experimental.pallas import tpu_sc as plsc

N_NODES = 10000
C = 128
E = 320000
EPS = 1e-5

NC, NS, L = 2, 16, 16
NW = NC * NS
CHUNK = 64
NCH_TOT = E // CHUNK
NCH_LO = NCH_TOT // NW
NCH_REM = NCH_TOT - NCH_LO * NW
NCH_MAX = NCH_LO + 1
STAGE = NCH_MAX * CHUNK
E_PAD = NCH_TOT * CHUNK + CHUNK
NBUF = 3
N_PAD = 10112
STRIPE = N_PAD // NS


_sc_mesh = plsc.VectorSubcoreMesh(
    core_axis_name="c", subcore_axis_name="s", num_cores=NC, num_subcores=NS
)


def _sc_body(xt, g_hbm, r_hbm, out_s, out_c, acc, cnt, gidx,
             rows0, rows1, rows2, rb0, rb1, rb2,
             gs0, gs1, gs2, ss0, ss1, ss2, is0, is1, is2):
    c = lax.axis_index("c")
    s = lax.axis_index("s")
    wid = c * NS + s
    ROWS = (rows0, rows1, rows2)
    RB = (rb0, rb1, rb2)
    GS = (gs0, gs1, gs2)
    SS = (ss0, ss1, ss2)
    IS = (is0, is1, is2)

    nch = NCH_LO + (wid < NCH_REM).astype(jnp.int32)
    ebase = (NCH_LO * wid + jnp.minimum(wid, NCH_REM)) * CHUNK

    pltpu.sync_copy(g_hbm.at[pl.ds(ebase, STAGE)], gidx)

    def i_start(a, b):
        pltpu.async_copy(r_hbm.at[pl.ds(ebase + a * CHUNK, CHUNK)], RB[b], IS[b])

    def i_wait(b):
        pltpu.make_async_copy(r_hbm.at[pl.ds(0, CHUNK)], RB[b], IS[b]).wait()

    def g_start(a, b):
        pltpu.async_copy(xt.at[gidx.at[pl.ds(a * CHUNK, CHUNK)]], ROWS[b], GS[b])

    def g_wait(b):
        pltpu.make_async_copy(xt.at[gidx.at[pl.ds(0, CHUNK)]], ROWS[b], GS[b]).wait()

    def s_wait(b):
        pltpu.make_async_copy(ROWS[b], acc.at[RB[b]], SS[b]).wait()

    i_start(0, 0)
    i_start(1, 1)
    g_start(0, 0)
    g_start(1, 1)

    zv = jnp.zeros((L,), jnp.float32)

    def zrow(i, carry):
        rows2[i // (C // L), pl.ds((i % (C // L)) * L, L)] = zv
        return carry

    lax.fori_loop(0, CHUNK * (C // L), zrow, 0)
    for k in range(STRIPE // CHUNK):
        pltpu.sync_copy(rows2, acc.at[pl.ds(s * STRIPE + k * CHUNK, CHUNK)])
    rem = STRIPE % CHUNK
    if rem:
        pltpu.sync_copy(
            rows2.at[pl.ds(0, rem)],
            acc.at[pl.ds(s * STRIPE + (STRIPE // CHUNK) * CHUNK, rem)],
        )

    def zcnt(i, carry):
        cnt[pl.ds(i * L, L)] = zv
        return carry

    lax.fori_loop(0, N_PAD // L, zcnt, 0)
    plsc.subcore_barrier()

    def counts(b):
        for k in range(CHUNK // L):
            v = RB[b][pl.ds(k * L, L)]
            r, last = plsc.scan_count(v)
            plsc.addupdate_scatter(cnt, [v], r.astype(jnp.float32), mask=last)

    def blocks(j, carry):
        for b in range(NBUF):
            a = NBUF * j + b

            @pl.when(a < nch)
            def _():
                g_wait(b)
                i_wait(b)
                pltpu.async_copy(ROWS[b], acc.at[RB[b]], SS[b], add=True)
                counts(b)
                b2 = (b + 2) % NBUF

                @pl.when(a + 2 < nch)
                def _():
                    @pl.when(a >= 1)
                    def _():
                        s_wait(b2)

                    i_start(a + 2, b2)
                    g_start(a + 2, b2)

        return carry

    lax.fori_loop(0, (NCH_MAX + NBUF - 1) // NBUF, blocks, 0)
    s_wait(0)
    s_wait(1)
    s_wait(2)

    plsc.subcore_barrier()
    pltpu.sync_copy(
        acc.at[pl.ds(s * STRIPE, STRIPE)],
        out_s.at[c, pl.ds(s * STRIPE, STRIPE)],
    )
    pltpu.sync_copy(cnt, out_c.at[c, s])


_sc_aggregate = functools.partial(
    pl.kernel,
    out_type=(
        jax.ShapeDtypeStruct((NC, N_PAD, C), jnp.float32),
        jax.ShapeDtypeStruct((NC, NS, N_PAD), jnp.float32),
    ),
    mesh=_sc_mesh,
    compiler_params=pltpu.CompilerParams(needs_layout_passes=False),
    scratch_types=[
        pltpu.VMEM_SHARED((N_PAD, C), jnp.float32),
        pltpu.VMEM((N_PAD,), jnp.float32),
        pltpu.VMEM((STAGE,), jnp.int32),
        pltpu.VMEM((CHUNK, C), jnp.float32),
        pltpu.VMEM((CHUNK, C), jnp.float32),
        pltpu.VMEM((CHUNK, C), jnp.float32),
        pltpu.VMEM((CHUNK,), jnp.int32),
        pltpu.VMEM((CHUNK,), jnp.int32),
        pltpu.VMEM((CHUNK,), jnp.int32),
        pltpu.SemaphoreType.DMA,
        pltpu.SemaphoreType.DMA,
        pltpu.SemaphoreType.DMA,
        pltpu.SemaphoreType.DMA,
        pltpu.SemaphoreType.DMA,
        pltpu.SemaphoreType.DMA,
        pltpu.SemaphoreType.DMA,
        pltpu.SemaphoreType.DMA,
        pltpu.SemaphoreType.DMA,
    ],
)(_sc_body)


def _tc_body(x_ref, p_ref, c_ref, wv_ref, wn_ref, g_ref, b_ref, a_ref, o_ref):
    x = x_ref[...]
    sums = p_ref[0, :N_NODES, :] + p_ref[1, :N_NODES, :]
    counts = jnp.sum(c_ref[...], axis=(0, 1))[:N_NODES]
    inv = 1.0 / jnp.maximum(counts, 1.0)
    zn = lax.dot_general(wn_ref[...], sums, (((1,), (1,)), ((), ())),
                         preferred_element_type=jnp.float32)
    zv = lax.dot_general(wv_ref[...], x, (((1,), (0,)), ((), ())),
                         preferred_element_type=jnp.float32)
    z = zv + zn * inv[None, :]
    mu = jnp.mean(z, axis=1, keepdims=True)
    var = jnp.mean(z * z, axis=1, keepdims=True) - mu * mu
    scale = g_ref[...] * lax.rsqrt(var + EPS)
    out = (z - mu) * scale + b_ref[...]
    alpha = a_ref[0, 0]
    o_ref[...] = jnp.where(out >= 0, out, alpha * out)


_tc_finish = pl.pallas_call(
    _tc_body,
    out_shape=jax.ShapeDtypeStruct((C, N_NODES), jnp.float32),
    in_specs=[
        pl.BlockSpec(memory_space=pltpu.VMEM),
        pl.BlockSpec(memory_space=pltpu.VMEM),
        pl.BlockSpec(memory_space=pltpu.VMEM),
        pl.BlockSpec(memory_space=pltpu.VMEM),
        pl.BlockSpec(memory_space=pltpu.VMEM),
        pl.BlockSpec(memory_space=pltpu.VMEM),
        pl.BlockSpec(memory_space=pltpu.VMEM),
        pl.BlockSpec(memory_space=pltpu.SMEM),
    ],
)


def kernel(in_features, W_v, W_n, gamma, beta, alpha, reduce_index, gather_index):
    x = in_features[0]
    xt = x.T
    gpad = jnp.concatenate(
        [gather_index, jnp.zeros((E_PAD - E,), jnp.int32)])
    psums, pcnts = _sc_aggregate(xt, gpad, reduce_index)
    out = _tc_finish(
        x, psums, pcnts, W_v, W_n,
        gamma.reshape(C, 1), beta.reshape(C, 1),
        jnp.reshape(alpha, (1, 1)),
    )
    return out[None]


# speedup vs baseline: 16.4912x; 1.0805x over previous
import functools

import jax
import jax.numpy as jnp
from jax import lax
from jax.experimental import pallas as pl
from jax.experimental.pallas import tpu as pltpu
from jax.experimental.pallas import tpu_sc as plsc

N_NODES = 10000
C = 128
E = 320000
EPS = 1e-5

NC, NS, L = 2, 16, 16
NW = NC * NS
EPT = E // NW
CHUNK = 80
NCHUNK = EPT // CHUNK
NBUF = 3
N_PAD = 10112
STRIPE = N_PAD // NS


_sc_mesh = plsc.VectorSubcoreMesh(
    core_axis_name="c", subcore_axis_name="s", num_cores=NC, num_subcores=NS
)


def _sc_body(xt, g_hbm, r_hbm, out_s, out_c, acc, cnt,
             rows0, rows1, rows2, gb0, gb1, gb2, rb0, rb1, rb2,
             gs0, gs1, gs2, ss0, ss1, ss2,
             ig0, ig1, ig2, ir0, ir1, ir2):
    c = lax.axis_index("c")
    s = lax.axis_index("s")
    wid = c * NS + s
    ROWS = (rows0, rows1, rows2)
    GB = (gb0, gb1, gb2)
    RB = (rb0, rb1, rb2)
    GS = (gs0, gs1, gs2)
    SS = (ss0, ss1, ss2)
    IG = (ig0, ig1, ig2)
    IR = (ir0, ir1, ir2)

    ebase = wid * EPT

    def ig_start(a, b):
        pltpu.async_copy(g_hbm.at[pl.ds(ebase + a * CHUNK, CHUNK)], GB[b], IG[b])

    def ig_wait(b):
        pltpu.make_async_copy(g_hbm.at[pl.ds(0, CHUNK)], GB[b], IG[b]).wait()

    def ir_start(a, b):
        pltpu.async_copy(r_hbm.at[pl.ds(ebase + a * CHUNK, CHUNK)], RB[b], IR[b])

    def ir_wait(b):
        pltpu.make_async_copy(r_hbm.at[pl.ds(0, CHUNK)], RB[b], IR[b]).wait()

    def g_start(b):
        pltpu.async_copy(xt.at[GB[b]], ROWS[b], GS[b])

    def g_wait(b):
        pltpu.make_async_copy(xt.at[GB[b]], ROWS[b], GS[b]).wait()

    def s_wait(b):
        pltpu.make_async_copy(ROWS[b], acc.at[RB[b]], SS[b]).wait()

    ig_start(0, 0)
    ig_start(1, 1)
    ir_start(0, 0)
    ir_start(1, 1)
    ig_wait(0)
    g_start(0)
    ig_wait(1)
    g_start(1)

    zv = jnp.zeros((L,), jnp.float32)

    def zrow(i, carry):
        rows2[i // (C // L), pl.ds((i % (C // L)) * L, L)] = zv
        return carry

    lax.fori_loop(0, CHUNK * (C // L), zrow, 0)
    for k in range(STRIPE // CHUNK):
        pltpu.sync_copy(rows2, acc.at[pl.ds(s * STRIPE + k * CHUNK, CHUNK)])
    rem = STRIPE % CHUNK
    if rem:
        pltpu.sync_copy(
            rows2.at[pl.ds(0, rem)],
            acc.at[pl.ds(s * STRIPE + (STRIPE // CHUNK) * CHUNK, rem)],
        )

    def zcnt(i, carry):
        cnt[pl.ds(i * L, L)] = zv
        return carry

    lax.fori_loop(0, N_PAD // L, zcnt, 0)
    plsc.subcore_barrier()

    def counts(b):
        for k in range(CHUNK // L):
            v = RB[b][pl.ds(k * L, L)]
            r, last = plsc.scan_count(v)
            plsc.addupdate_scatter(cnt, [v], r.astype(jnp.float32), mask=last)

    def blocks(j, carry):
        for b in range(NBUF):
            a = NBUF * j + b

            @pl.when(a < NCHUNK)
            def _():
                b2 = (b + 2) % NBUF

                @pl.when(a + 2 < NCHUNK)
                def _():
                    ig_start(a + 2, b2)

                g_wait(b)
                ir_wait(b)
                pltpu.async_copy(ROWS[b], acc.at[RB[b]], SS[b], add=True)

                @pl.when(a + 2 < NCHUNK)
                def _():
                    @pl.when(a >= 1)
                    def _():
                        s_wait(b2)

                    ir_start(a + 2, b2)
                    ig_wait(b2)
                    g_start(b2)

                counts(b)

        return carry

    lax.fori_loop(0, (NCHUNK + NBUF - 1) // NBUF, blocks, 0)
    s_wait(0)
    s_wait(1)
    s_wait(2)

    plsc.subcore_barrier()
    pltpu.sync_copy(
        acc.at[pl.ds(s * STRIPE, STRIPE)],
        out_s.at[c, pl.ds(s * STRIPE, STRIPE)],
    )
    pltpu.sync_copy(cnt, out_c.at[c, s])


_sc_aggregate = functools.partial(
    pl.kernel,
    out_type=(
        jax.ShapeDtypeStruct((NC, N_PAD, C), jnp.float32),
        jax.ShapeDtypeStruct((NC, NS, N_PAD), jnp.float32),
    ),
    mesh=_sc_mesh,
    compiler_params=pltpu.CompilerParams(needs_layout_passes=False),
    scratch_types=[
        pltpu.VMEM_SHARED((N_PAD, C), jnp.float32),
        pltpu.VMEM((N_PAD,), jnp.float32),
        pltpu.VMEM((CHUNK, C), jnp.float32),
        pltpu.VMEM((CHUNK, C), jnp.float32),
        pltpu.VMEM((CHUNK, C), jnp.float32),
        pltpu.VMEM((CHUNK,), jnp.int32),
        pltpu.VMEM((CHUNK,), jnp.int32),
        pltpu.VMEM((CHUNK,), jnp.int32),
        pltpu.VMEM((CHUNK,), jnp.int32),
        pltpu.VMEM((CHUNK,), jnp.int32),
        pltpu.VMEM((CHUNK,), jnp.int32),
        pltpu.SemaphoreType.DMA,
        pltpu.SemaphoreType.DMA,
        pltpu.SemaphoreType.DMA,
        pltpu.SemaphoreType.DMA,
        pltpu.SemaphoreType.DMA,
        pltpu.SemaphoreType.DMA,
        pltpu.SemaphoreType.DMA,
        pltpu.SemaphoreType.DMA,
        pltpu.SemaphoreType.DMA,
        pltpu.SemaphoreType.DMA,
        pltpu.SemaphoreType.DMA,
        pltpu.SemaphoreType.DMA,
    ],
)(_sc_body)


def _tc_body(x_ref, p_ref, c_ref, wv_ref, wn_ref, g_ref, b_ref, a_ref, o_ref):
    x = x_ref[...]
    sums = p_ref[0, :N_NODES, :] + p_ref[1, :N_NODES, :]
    counts = jnp.sum(c_ref[...], axis=(0, 1))[:N_NODES]
    inv = 1.0 / jnp.maximum(counts, 1.0)
    zn = lax.dot_general(wn_ref[...], sums, (((1,), (1,)), ((), ())),
                         preferred_element_type=jnp.float32)
    zv = lax.dot_general(wv_ref[...], x, (((1,), (0,)), ((), ())),
                         preferred_element_type=jnp.float32)
    z = zv + zn * inv[None, :]
    mu = jnp.mean(z, axis=1, keepdims=True)
    var = jnp.mean(z * z, axis=1, keepdims=True) - mu * mu
    scale = g_ref[...] * lax.rsqrt(var + EPS)
    out = (z - mu) * scale + b_ref[...]
    alpha = a_ref[0, 0]
    o_ref[...] = jnp.where(out >= 0, out, alpha * out)


_tc_finish = pl.pallas_call(
    _tc_body,
    out_shape=jax.ShapeDtypeStruct((C, N_NODES), jnp.float32),
    in_specs=[
        pl.BlockSpec(memory_space=pltpu.VMEM),
        pl.BlockSpec(memory_space=pltpu.VMEM),
        pl.BlockSpec(memory_space=pltpu.VMEM),
        pl.BlockSpec(memory_space=pltpu.VMEM),
        pl.BlockSpec(memory_space=pltpu.VMEM),
        pl.BlockSpec(memory_space=pltpu.VMEM),
        pl.BlockSpec(memory_space=pltpu.VMEM),
        pl.BlockSpec(memory_space=pltpu.SMEM),
    ],
)


def kernel(in_features, W_v, W_n, gamma, beta, alpha, reduce_index, gather_index):
    x = in_features[0]
    xt = x.T
    psums, pcnts = _sc_aggregate(xt, gather_index, reduce_index)
    out = _tc_finish(
        x, psums, pcnts, W_v, W_n,
        gamma.reshape(C, 1), beta.reshape(C, 1),
        jnp.reshape(alpha, (1, 1)),
    )
    return out[None]
